# Initial kernel scaffold; baseline (speedup 1.0000x reference)
import jax
import jax.numpy as jnp
from jax.experimental import pallas as pl


def kernel(speaker, x, edge_index, edge_norm, edge_type, seq_lengths, umask, w1, w2, W_lin, b_lin, W_fc, b_fc):
    raise NotImplementedError("write your pallas kernel here")


# speedup vs baseline: 7.7324x; 7.7324x over previous
import functools

import jax
import jax.numpy as jnp
from jax import lax
from jax.experimental import pallas as pl
from jax.experimental.pallas import tpu as pltpu
from jax.experimental.pallas import tpu_sc as plsc

N = 10000
E = 160000
D = 128
R = 8
C = 7

NC = 2
NS = 16
L = 16
NW = NC * NS
CH = 128
EPT = 5120
MAXCH = EPT // CH
ROWS_PT = 624


def _sc_mesh():
    return plsc.VectorSubcoreMesh(core_axis_name="c", subcore_axis_name="s")


def _zero_spmem_slices(src_v, dst_s, sid):
    r0 = sid * ROWS_PT
    ofs = 0
    while ofs < ROWS_PT:
        sz = min(CH, ROWS_PT - ofs)
        pltpu.sync_copy(src_v.at[pl.ds(0, sz)], dst_s.at[pl.ds(r0 + ofs, sz)])
        ofs += sz

    @pl.when(sid == NS - 1)
    def _():
        pltpu.sync_copy(src_v.at[pl.ds(0, N - NS * ROWS_PT)],
                        dst_s.at[pl.ds(NS * ROWS_PT, N - NS * ROWS_PT)])


def _write_spmem_slices(src_s, dst_h, cid, sid):
    r0 = sid * ROWS_PT
    pltpu.sync_copy(src_s.at[pl.ds(r0, ROWS_PT)],
                    dst_h.at[cid, pl.ds(r0, ROWS_PT)])

    @pl.when(sid == NS - 1)
    def _():
        pltpu.sync_copy(src_s.at[pl.ds(NS * ROWS_PT, N - NS * ROWS_PT)],
                        dst_h.at[cid, pl.ds(NS * ROWS_PT, N - NS * ROWS_PT)])


def _sc_agg_body(table_h, ei_h, et_h, en_h, out_h,
                 acc_s, i_v, j_v, t_v, g_v, en_v, rows_v, sem):
    cid = lax.axis_index("c")
    sid = lax.axis_index("s")
    wid = sid * NC + cid

    def init_body(rr, _):
        z16 = jnp.zeros((L,), jnp.float32)
        for d8 in range(D // L):
            rows_v[rr, pl.ds(d8 * L, L)] = z16
        return 0

    lax.fori_loop(0, CH, init_body, 0)
    _zero_spmem_slices(rows_v, acc_s, sid)
    plsc.subcore_barrier()

    base = wid * EPT
    nch = jnp.minimum((E - base) // CH, MAXCH)

    def chunk_body(ci, _):
        off = base + ci * CH
        pltpu.sync_copy(ei_h.at[0, pl.ds(off, CH)], i_v)
        pltpu.sync_copy(ei_h.at[1, pl.ds(off, CH)], j_v)
        pltpu.sync_copy(et_h.at[pl.ds(off, CH)], t_v)
        pltpu.sync_copy(en_h.at[pl.ds(off, CH)], en_v)

        def gidx_body(k, _):
            s = pl.ds(k * L, L)
            g_v[s] = t_v[s] * N + j_v[s]
            return 0

        lax.fori_loop(0, CH // L, gidx_body, 0)

        pltpu.async_copy(table_h.at[g_v], rows_v, sem).wait()

        def scale_body(k, _):
            env = en_v[pl.ds(k * L, L)] * 2.0
            for lane in range(L):
                sv = jnp.full((L,), env[lane], jnp.float32)
                e = k * L + lane
                for d8 in range(D // L):
                    s = pl.ds(d8 * L, L)
                    rows_v[e, s] = rows_v[e, s] * sv
            return 0

        lax.fori_loop(0, CH // L, scale_body, 0)

        pltpu.sync_copy(rows_v, acc_s.at[i_v], add=True)
        return 0

    lax.fori_loop(0, nch, chunk_body, 0)
    plsc.subcore_barrier()

    _write_spmem_slices(acc_s, out_h, cid, sid)


@functools.lru_cache(maxsize=None)
def _sc_aggregate_kernel():
    return pl.kernel(
        _sc_agg_body,
        out_type=jax.ShapeDtypeStruct((NC, N, D), jnp.float32),
        mesh=_sc_mesh(),
        scratch_types=[
        pltpu.VMEM_SHARED((N, D), jnp.float32),
        pltpu.VMEM((CH,), jnp.int32),
        pltpu.VMEM((CH,), jnp.int32),
        pltpu.VMEM((CH,), jnp.int32),
        pltpu.VMEM((CH,), jnp.int32),
        pltpu.VMEM((CH,), jnp.float32),
        pltpu.VMEM((CH, D), jnp.float32),
        pltpu.SemaphoreType.DMA,
        ],
    )


def _sc_cnt_body(ei_h, cnt_h, cnt_s, i_v, ones_v, zl_v):
    cid = lax.axis_index("c")
    sid = lax.axis_index("s")
    wid = sid * NC + cid

    def init_body(rr, _):
        for d8 in range(D // L):
            s = pl.ds(d8 * L, L)
            ones_v[rr, s] = jnp.ones((L,), jnp.float32)
            zl_v[rr, s] = jnp.zeros((L,), jnp.float32)
        return 0

    lax.fori_loop(0, CH, init_body, 0)
    _zero_spmem_slices(zl_v, cnt_s, sid)
    plsc.subcore_barrier()

    base = wid * EPT
    nch = jnp.minimum((E - base) // CH, MAXCH)

    def chunk_body(ci, _):
        off = base + ci * CH
        pltpu.sync_copy(ei_h.at[0, pl.ds(off, CH)], i_v)
        pltpu.sync_copy(ones_v, cnt_s.at[i_v], add=True)
        return 0

    lax.fori_loop(0, nch, chunk_body, 0)
    plsc.subcore_barrier()

    _write_spmem_slices(cnt_s, cnt_h, cid, sid)


@functools.lru_cache(maxsize=None)
def _sc_count_kernel():
    return pl.kernel(
        _sc_cnt_body,
        out_type=jax.ShapeDtypeStruct((NC, N, D), jnp.float32),
        mesh=_sc_mesh(),
        scratch_types=[
            pltpu.VMEM_SHARED((N, D), jnp.float32),
            pltpu.VMEM((CH,), jnp.int32),
            pltpu.VMEM((CH, D), jnp.float32),
            pltpu.VMEM((CH, D), jnp.float32),
        ],
    )


def _tc_transform(x, w):
    bn = 1000

    def tbody(x_ref, w_ref, o_ref):
        o_ref[0] = jnp.dot(x_ref[...], w_ref[0],
                           preferred_element_type=jnp.float32)

    return pl.pallas_call(
        tbody,
        grid=(R, N // bn),
        in_specs=[pl.BlockSpec((bn, D), lambda r, n: (n, 0)),
                  pl.BlockSpec((1, D, D), lambda r, n: (r, 0, 0))],
        out_specs=pl.BlockSpec((1, bn, D), lambda r, n: (r, n, 0)),
        out_shape=jax.ShapeDtypeStruct((R, N, D), jnp.float32),
    )(x, w)


def _tc_norm_transform(p, cnt, w):
    bn = 1000

    def nbody(p_ref, c_ref, w_ref, o_ref):
        c = c_ref[0, :, 0:1] + c_ref[1, :, 0:1]
        h = jax.nn.sigmoid((p_ref[0] + p_ref[1]) / jnp.maximum(c, 1.0))
        for r in range(R):
            o_ref[r] = jnp.dot(h, w_ref[r], preferred_element_type=jnp.float32)

    return pl.pallas_call(
        nbody,
        grid=(N // bn,),
        in_specs=[pl.BlockSpec((2, bn, D), lambda n: (0, n, 0)),
                  pl.BlockSpec((2, bn, D), lambda n: (0, n, 0)),
                  pl.BlockSpec((R, D, D), lambda n: (0, 0, 0))],
        out_specs=pl.BlockSpec((R, bn, D), lambda n: (0, n, 0)),
        out_shape=jax.ShapeDtypeStruct((R, N, D), jnp.float32),
    )(p, cnt, w)


def _tc_head(p2, cnt, x, wl, bl, wf, bf):
    bn = 1000

    def hbody(p_ref, c_ref, x_ref, wl_ref, bl_ref, wf_ref, bf_ref,
              lp_ref, em_ref):
        c = c_ref[0, :, 0:1] + c_ref[1, :, 0:1]
        agg = (p_ref[0] + p_ref[1]) / jnp.maximum(c, 1.0)
        em = jnp.concatenate([x_ref[...], agg], axis=1)
        em_ref[...] = em
        h = lax.dot_general(em, wl_ref[...], (((1,), (1,)), ((), ())),
                            preferred_element_type=jnp.float32) + bl_ref[...]
        h = jnp.maximum(h, 0.0)
        logits = lax.dot_general(h, wf_ref[...], (((1,), (1,)), ((), ())),
                                 preferred_element_type=jnp.float32) + bf_ref[...]
        m = jnp.max(logits, axis=1, keepdims=True)
        z = logits - m
        lp_ref[...] = z - jnp.log(jnp.sum(jnp.exp(z), axis=1, keepdims=True))

    return pl.pallas_call(
        hbody,
        grid=(N // bn,),
        in_specs=[pl.BlockSpec((2, bn, D), lambda n: (0, n, 0)),
                  pl.BlockSpec((2, bn, D), lambda n: (0, n, 0)),
                  pl.BlockSpec((bn, D), lambda n: (n, 0)),
                  pl.BlockSpec((D, 2 * D), lambda n: (0, 0)),
                  pl.BlockSpec((1, D), lambda n: (0, 0)),
                  pl.BlockSpec((C, D), lambda n: (0, 0)),
                  pl.BlockSpec((1, C), lambda n: (0, 0))],
        out_specs=[pl.BlockSpec((bn, C), lambda n: (n, 0)),
                   pl.BlockSpec((bn, 2 * D), lambda n: (n, 0))],
        out_shape=[jax.ShapeDtypeStruct((N, C), jnp.float32),
                   jax.ShapeDtypeStruct((N, 2 * D), jnp.float32)],
    )(p2, cnt, x, wl, bl, wf, bf)


def kernel(speaker, x, edge_index, edge_norm, edge_type, seq_lengths, umask,
           w1, w2, W_lin, b_lin, W_fc, b_fc):
    ei = edge_index.astype(jnp.int32)
    et = edge_type.astype(jnp.int32)
    en = edge_norm.astype(jnp.float32)

    y1 = _tc_transform(x, w1).reshape(R * N, D)
    cnt = _sc_count_kernel()(ei)
    p1 = _sc_aggregate_kernel()(y1, ei, et, en)
    y2 = _tc_norm_transform(p1, cnt, w2).reshape(R * N, D)
    p2 = _sc_aggregate_kernel()(y2, ei, et, en)
    log_prob, emotions = _tc_head(p2, cnt, x, W_lin,
                                  b_lin.reshape(1, D), W_fc,
                                  b_fc.reshape(1, C))
    return (log_prob, x, emotions)


# speedup vs baseline: 12.8170x; 1.6576x over previous
import functools

import jax
import jax.numpy as jnp
from jax import lax
from jax.experimental import pallas as pl
from jax.experimental.pallas import tpu as pltpu
from jax.experimental.pallas import tpu_sc as plsc

N = 10000
E = 160000
D = 128
R = 8
C = 7

NC = 2
NS = 16
L = 16
NW = NC * NS
CL = 16
CH = 128
EPT = 5120
MAXCH = EPT // CH
ROWS_PT = 624


def _sc_mesh():
    return plsc.VectorSubcoreMesh(core_axis_name="c", subcore_axis_name="s")


def _zero_spmem_slices(src_v, dst_s, sid):
    r0 = sid * ROWS_PT
    ofs = 0
    while ofs < ROWS_PT:
        sz = min(CH, ROWS_PT - ofs)
        pltpu.sync_copy(src_v.at[pl.ds(0, sz)], dst_s.at[pl.ds(r0 + ofs, sz)])
        ofs += sz

    @pl.when(sid == NS - 1)
    def _():
        pltpu.sync_copy(src_v.at[pl.ds(0, N - NS * ROWS_PT)],
                        dst_s.at[pl.ds(NS * ROWS_PT, N - NS * ROWS_PT)])


def _write_spmem_slices(src_s, dst_h, cid, sid):
    r0 = sid * ROWS_PT
    pltpu.sync_copy(src_s.at[pl.ds(r0, ROWS_PT)],
                    dst_h.at[cid, pl.ds(r0, ROWS_PT)])

    @pl.when(sid == NS - 1)
    def _():
        pltpu.sync_copy(src_s.at[pl.ds(NS * ROWS_PT, N - NS * ROWS_PT)],
                        dst_h.at[cid, pl.ds(NS * ROWS_PT, N - NS * ROWS_PT)])


def _sc_agg_body(table_h, ei_h, et_h, en_h, out_h,
                 acc_s,
                 i0, j0, t0, g0, en0, r0,
                 i1, j1, t1, g1, en1, r1,
                 gs0, gs1, is0, is1):
    cid = lax.axis_index("c")
    sid = lax.axis_index("s")
    wid = sid * NC + cid
    base = wid * EPT
    nch = jnp.minimum((E - base) // CH, MAXCH)

    bufs = ((i0, j0, t0, g0, en0, r0, gs0, is0),
            (i1, j1, t1, g1, en1, r1, gs1, is1))

    def init_body(rr, _):
        z16 = jnp.zeros((L,), jnp.float32)
        for d8 in range(D // L):
            r0[rr, pl.ds(d8 * L, L)] = z16
        return 0

    lax.fori_loop(0, CH, init_body, 0)
    _zero_spmem_slices(r0, acc_s, sid)
    plsc.subcore_barrier()

    def idx_start(c, b):
        i_v, j_v, t_v, g_v, en_v, rows_v, gsem, isem = bufs[b]
        off = base + c * CH
        pltpu.async_copy(ei_h.at[0, pl.ds(off, CH)], i_v, isem)
        pltpu.async_copy(ei_h.at[1, pl.ds(off, CH)], j_v, isem)
        pltpu.async_copy(et_h.at[pl.ds(off, CH)], t_v, isem)
        pltpu.async_copy(en_h.at[pl.ds(off, CH)], en_v, isem)

    def gather_start(c, b):
        i_v, j_v, t_v, g_v, en_v, rows_v, gsem, isem = bufs[b]
        off = base + c * CH
        pltpu.make_async_copy(ei_h.at[0, pl.ds(off, CH)], i_v, isem).wait()
        pltpu.make_async_copy(ei_h.at[1, pl.ds(off, CH)], j_v, isem).wait()
        pltpu.make_async_copy(et_h.at[pl.ds(off, CH)], t_v, isem).wait()
        pltpu.make_async_copy(en_h.at[pl.ds(off, CH)], en_v, isem).wait()

        def gidx_body(k, _):
            s = pl.ds(k * L, L)
            g_v[s] = t_v[s] * N + j_v[s]
            return 0

        lax.fori_loop(0, CH // L, gidx_body, 0)
        pltpu.async_copy(table_h.at[g_v], rows_v, gsem)

    def scale_scatter(b):
        i_v, j_v, t_v, g_v, en_v, rows_v, gsem, isem = bufs[b]
        pltpu.make_async_copy(table_h.at[g_v], rows_v, gsem).wait()

        def scale_body(k, _):
            env = en_v[pl.ds(k * L, L)] * 2.0
            for lane in range(L):
                sv = jnp.full((L,), env[lane], jnp.float32)
                e = k * L + lane
                for d8 in range(D // L):
                    s = pl.ds(d8 * L, L)
                    rows_v[e, s] = rows_v[e, s] * sv
            return 0

        lax.fori_loop(0, CH // L, scale_body, 0)
        pltpu.sync_copy(rows_v, acc_s.at[i_v], add=True)

    idx_start(0, 0)
    idx_start(1, 1)
    gather_start(0, 0)

    def iter_body(k, _):
        c = 2 * k
        gather_start(c + 1, 1)
        scale_scatter(0)

        @pl.when(c + 2 < nch)
        def _():
            idx_start(c + 2, 0)
            gather_start(c + 2, 0)

        scale_scatter(1)

        @pl.when(c + 3 < nch)
        def _():
            idx_start(c + 3, 1)

        return 0

    lax.fori_loop(0, nch // 2, iter_body, 0)

    @pl.when(nch % 2 == 1)
    def _():
        scale_scatter(0)

    plsc.subcore_barrier()
    _write_spmem_slices(acc_s, out_h, cid, sid)


@functools.lru_cache(maxsize=None)
def _sc_aggregate_kernel():
    scratch = [pltpu.VMEM_SHARED((N, D), jnp.float32)]
    for _ in range(2):
        scratch += [
            pltpu.VMEM((CH,), jnp.int32),
            pltpu.VMEM((CH,), jnp.int32),
            pltpu.VMEM((CH,), jnp.int32),
            pltpu.VMEM((CH,), jnp.int32),
            pltpu.VMEM((CH,), jnp.float32),
            pltpu.VMEM((CH, D), jnp.float32),
        ]
    scratch += [pltpu.SemaphoreType.DMA] * 4
    return pl.kernel(
        _sc_agg_body,
        out_type=jax.ShapeDtypeStruct((NC, N, D), jnp.float32),
        mesh=_sc_mesh(),
        scratch_types=scratch,
    )


def _sc_cnt_body(ei_h, cnt_h, cnt_s, i_v, ones_v, zl_v):
    cid = lax.axis_index("c")
    sid = lax.axis_index("s")
    wid = sid * NC + cid

    def init_body(rr, _):
        for d8 in range(D // L):
            s = pl.ds(d8 * L, L)
            ones_v[rr, s] = jnp.ones((L,), jnp.float32)
            zl_v[rr, s] = jnp.zeros((L,), jnp.float32)
        return 0

    lax.fori_loop(0, CH, init_body, 0)
    _zero_spmem_slices(zl_v, cnt_s, sid)
    plsc.subcore_barrier()

    base = wid * EPT
    nch = jnp.minimum((E - base) // CH, MAXCH)

    def chunk_body(ci, _):
        off = base + ci * CH
        pltpu.sync_copy(ei_h.at[0, pl.ds(off, CH)], i_v)
        pltpu.sync_copy(ones_v, cnt_s.at[i_v], add=True)
        return 0

    lax.fori_loop(0, nch, chunk_body, 0)
    plsc.subcore_barrier()

    _write_spmem_slices(cnt_s, cnt_h, cid, sid)


@functools.lru_cache(maxsize=None)
def _sc_count_kernel():
    return pl.kernel(
        _sc_cnt_body,
        out_type=jax.ShapeDtypeStruct((NC, N, D), jnp.float32),
        mesh=_sc_mesh(),
        scratch_types=[
            pltpu.VMEM_SHARED((N, D), jnp.float32),
            pltpu.VMEM((CH,), jnp.int32),
            pltpu.VMEM((CH, D), jnp.float32),
            pltpu.VMEM((CH, D), jnp.float32),
        ],
    )


def _tc_transform(x, w):
    bn = 1000

    def tbody(x_ref, w_ref, o_ref):
        o_ref[0] = jnp.dot(x_ref[...], w_ref[0],
                           preferred_element_type=jnp.float32)

    return pl.pallas_call(
        tbody,
        grid=(R, N // bn),
        in_specs=[pl.BlockSpec((bn, D), lambda r, n: (n, 0)),
                  pl.BlockSpec((1, D, D), lambda r, n: (r, 0, 0))],
        out_specs=pl.BlockSpec((1, bn, D), lambda r, n: (r, n, 0)),
        out_shape=jax.ShapeDtypeStruct((R, N, D), jnp.float32),
    )(x, w)


def _tc_norm_transform(p, cnt, w):
    bn = 1000

    def nbody(p_ref, c_ref, w_ref, o_ref):
        c = c_ref[0, :, 0:1] + c_ref[1, :, 0:1]
        h = jax.nn.sigmoid((p_ref[0] + p_ref[1]) / jnp.maximum(c, 1.0))
        for r in range(R):
            o_ref[r] = jnp.dot(h, w_ref[r], preferred_element_type=jnp.float32)

    return pl.pallas_call(
        nbody,
        grid=(N // bn,),
        in_specs=[pl.BlockSpec((2, bn, D), lambda n: (0, n, 0)),
                  pl.BlockSpec((2, bn, D), lambda n: (0, n, 0)),
                  pl.BlockSpec((R, D, D), lambda n: (0, 0, 0))],
        out_specs=pl.BlockSpec((R, bn, D), lambda n: (0, n, 0)),
        out_shape=jax.ShapeDtypeStruct((R, N, D), jnp.float32),
    )(p, cnt, w)


def _tc_head(p2, cnt, x, wl, bl, wf, bf):
    bn = 1000

    def hbody(p_ref, c_ref, x_ref, wl_ref, bl_ref, wf_ref, bf_ref,
              lp_ref, em_ref):
        c = c_ref[0, :, 0:1] + c_ref[1, :, 0:1]
        agg = (p_ref[0] + p_ref[1]) / jnp.maximum(c, 1.0)
        em = jnp.concatenate([x_ref[...], agg], axis=1)
        em_ref[...] = em
        h = lax.dot_general(em, wl_ref[...], (((1,), (1,)), ((), ())),
                            preferred_element_type=jnp.float32) + bl_ref[...]
        h = jnp.maximum(h, 0.0)
        logits = lax.dot_general(h, wf_ref[...], (((1,), (1,)), ((), ())),
                                 preferred_element_type=jnp.float32) + bf_ref[...]
        m = jnp.max(logits, axis=1, keepdims=True)
        z = logits - m
        lp_ref[...] = z - jnp.log(jnp.sum(jnp.exp(z), axis=1, keepdims=True))

    return pl.pallas_call(
        hbody,
        grid=(N // bn,),
        in_specs=[pl.BlockSpec((2, bn, D), lambda n: (0, n, 0)),
                  pl.BlockSpec((2, bn, D), lambda n: (0, n, 0)),
                  pl.BlockSpec((bn, D), lambda n: (n, 0)),
                  pl.BlockSpec((D, 2 * D), lambda n: (0, 0)),
                  pl.BlockSpec((1, D), lambda n: (0, 0)),
                  pl.BlockSpec((C, D), lambda n: (0, 0)),
                  pl.BlockSpec((1, C), lambda n: (0, 0))],
        out_specs=[pl.BlockSpec((bn, C), lambda n: (n, 0)),
                   pl.BlockSpec((bn, 2 * D), lambda n: (n, 0))],
        out_shape=[jax.ShapeDtypeStruct((N, C), jnp.float32),
                   jax.ShapeDtypeStruct((N, 2 * D), jnp.float32)],
    )(p2, cnt, x, wl, bl, wf, bf)


def kernel(speaker, x, edge_index, edge_norm, edge_type, seq_lengths, umask,
           w1, w2, W_lin, b_lin, W_fc, b_fc):
    ei = edge_index.astype(jnp.int32)
    et = edge_type.astype(jnp.int32)
    en = edge_norm.astype(jnp.float32)

    y1 = _tc_transform(x, w1).reshape(R * N, D)
    cnt = _sc_count_kernel()(ei)
    p1 = _sc_aggregate_kernel()(y1, ei, et, en)
    y2 = _tc_norm_transform(p1, cnt, w2).reshape(R * N, D)
    p2 = _sc_aggregate_kernel()(y2, ei, et, en)
    log_prob, emotions = _tc_head(p2, cnt, x, W_lin,
                                  b_lin.reshape(1, D), W_fc,
                                  b_fc.reshape(1, C))
    return (log_prob, x, emotions)


# speedup vs baseline: 14.2419x; 1.1112x over previous
import functools

import jax
import jax.numpy as jnp
from jax import lax
from jax.experimental import pallas as pl
from jax.experimental.pallas import tpu as pltpu
from jax.experimental.pallas import tpu_sc as plsc

N = 10000
E = 160000
D = 128
R = 8
C = 7

NC = 2
NS = 16
L = 16
NW = NC * NS
CH = 128
EPT = 5120
MAXCH = EPT // CH
ROWS_PT = 624


def _sc_mesh():
    return plsc.VectorSubcoreMesh(core_axis_name="c", subcore_axis_name="s")


def _zero_spmem_slices(src_v, dst_s, sid):
    r0 = sid * ROWS_PT
    ofs = 0
    while ofs < ROWS_PT:
        sz = min(CH, ROWS_PT - ofs)
        pltpu.sync_copy(src_v.at[pl.ds(0, sz)], dst_s.at[pl.ds(r0 + ofs, sz)])
        ofs += sz

    @pl.when(sid == NS - 1)
    def _():
        pltpu.sync_copy(src_v.at[pl.ds(0, N - NS * ROWS_PT)],
                        dst_s.at[pl.ds(NS * ROWS_PT, N - NS * ROWS_PT)])


def _write_spmem_slices(src_s, dst_h, cid, sid):
    r0 = sid * ROWS_PT
    pltpu.sync_copy(src_s.at[pl.ds(r0, ROWS_PT)],
                    dst_h.at[cid, pl.ds(r0, ROWS_PT)])

    @pl.when(sid == NS - 1)
    def _():
        pltpu.sync_copy(src_s.at[pl.ds(NS * ROWS_PT, N - NS * ROWS_PT)],
                        dst_h.at[cid, pl.ds(NS * ROWS_PT, N - NS * ROWS_PT)])


def _vcopy(src_v, dst_v):
    def body(k, _):
        s = pl.ds(k * L, L)
        dst_v[s] = src_v[s]
        return 0

    lax.fori_loop(0, CH // L, body, 0)


def _sc_agg_body(table_h, ei_h, et_h, en_h, out_h,
                 acc_s,
                 i0, j0, t0, g0, en0, r0, si0, sen0,
                 i1, j1, t1, g1, en1, r1, si1, sen1,
                 gs0, gs1, is0, is1):
    cid = lax.axis_index("c")
    sid = lax.axis_index("s")
    wid = sid * NC + cid
    base = wid * EPT
    nch = jnp.minimum((E - base) // CH, MAXCH)

    bufs = ((i0, j0, t0, g0, en0, r0, si0, sen0, gs0, is0),
            (i1, j1, t1, g1, en1, r1, si1, sen1, gs1, is1))

    def init_body(rr, _):
        z16 = jnp.zeros((L,), jnp.float32)
        for d8 in range(D // L):
            r0[rr, pl.ds(d8 * L, L)] = z16
        return 0

    lax.fori_loop(0, CH, init_body, 0)
    _zero_spmem_slices(r0, acc_s, sid)
    plsc.subcore_barrier()

    def idx_start(c, b):
        i_v, j_v, t_v, g_v, en_v, rows_v, si_v, sen_v, gsem, isem = bufs[b]
        off = base + c * CH
        pltpu.async_copy(ei_h.at[0, pl.ds(off, CH)], i_v, isem)
        pltpu.async_copy(ei_h.at[1, pl.ds(off, CH)], j_v, isem)
        pltpu.async_copy(et_h.at[pl.ds(off, CH)], t_v, isem)
        pltpu.async_copy(en_h.at[pl.ds(off, CH)], en_v, isem)

    def gather_start(c, b):
        i_v, j_v, t_v, g_v, en_v, rows_v, si_v, sen_v, gsem, isem = bufs[b]
        off = base + c * CH
        pltpu.make_async_copy(ei_h.at[0, pl.ds(off, CH)], i_v, isem).wait()
        pltpu.make_async_copy(ei_h.at[1, pl.ds(off, CH)], j_v, isem).wait()
        pltpu.make_async_copy(et_h.at[pl.ds(off, CH)], t_v, isem).wait()
        pltpu.make_async_copy(en_h.at[pl.ds(off, CH)], en_v, isem).wait()

        def gidx_body(k, _):
            s = pl.ds(k * L, L)
            g_v[s] = t_v[s] * N + j_v[s]
            return 0

        lax.fori_loop(0, CH // L, gidx_body, 0)
        pltpu.async_copy(table_h.at[g_v], rows_v, gsem)

    def save_idx(b):
        i_v, j_v, t_v, g_v, en_v, rows_v, si_v, sen_v, gsem, isem = bufs[b]
        _vcopy(i_v, si_v)
        _vcopy(en_v, sen_v)

    def scale_scatter(b):
        i_v, j_v, t_v, g_v, en_v, rows_v, si_v, sen_v, gsem, isem = bufs[b]
        pltpu.make_async_copy(table_h.at[g_v], rows_v, gsem).wait()

        def scale_body(k, _):
            env = sen_v[pl.ds(k * L, L)] * 2.0
            for lane in range(L):
                sv = jnp.full((L,), env[lane], jnp.float32)
                e = k * L + lane
                for d8 in range(D // L):
                    s = pl.ds(d8 * L, L)
                    rows_v[e, s] = rows_v[e, s] * sv
            return 0

        lax.fori_loop(0, CH // L, scale_body, 0)
        pltpu.sync_copy(rows_v, acc_s.at[si_v], add=True)

    idx_start(0, 0)
    idx_start(1, 1)
    gather_start(0, 0)

    def iter_body(k, _):
        c = 2 * k
        gather_start(c + 1, 1)
        save_idx(0)

        @pl.when(c + 2 < nch)
        def _():
            idx_start(c + 2, 0)

        scale_scatter(0)

        @pl.when(c + 2 < nch)
        def _():
            gather_start(c + 2, 0)

        save_idx(1)

        @pl.when(c + 3 < nch)
        def _():
            idx_start(c + 3, 1)

        scale_scatter(1)
        return 0

    lax.fori_loop(0, nch // 2, iter_body, 0)

    @pl.when(nch % 2 == 1)
    def _():
        save_idx(0)
        scale_scatter(0)

    plsc.subcore_barrier()
    _write_spmem_slices(acc_s, out_h, cid, sid)


@functools.lru_cache(maxsize=None)
def _sc_aggregate_kernel():
    scratch = [pltpu.VMEM_SHARED((N, D), jnp.float32)]
    for _ in range(2):
        scratch += [
            pltpu.VMEM((CH,), jnp.int32),
            pltpu.VMEM((CH,), jnp.int32),
            pltpu.VMEM((CH,), jnp.int32),
            pltpu.VMEM((CH,), jnp.int32),
            pltpu.VMEM((CH,), jnp.float32),
            pltpu.VMEM((CH, D), jnp.float32),
            pltpu.VMEM((CH,), jnp.int32),
            pltpu.VMEM((CH,), jnp.float32),
        ]
    scratch += [pltpu.SemaphoreType.DMA] * 4
    return pl.kernel(
        _sc_agg_body,
        out_type=jax.ShapeDtypeStruct((NC, N, D), jnp.float32),
        mesh=_sc_mesh(),
        scratch_types=scratch,
    )


def _sc_cnt_body(ei_h, cnt_h, cnt_s, i0, i1, ones_v, zl_v, is0, is1):
    cid = lax.axis_index("c")
    sid = lax.axis_index("s")
    wid = sid * NC + cid
    base = wid * EPT
    nch = jnp.minimum((E - base) // CH, MAXCH)

    bufs = ((i0, is0), (i1, is1))

    def init_body(rr, _):
        for d8 in range(D // L):
            s = pl.ds(d8 * L, L)
            ones_v[rr, s] = jnp.ones((L,), jnp.float32)
            zl_v[rr, s] = jnp.zeros((L,), jnp.float32)
        return 0

    lax.fori_loop(0, CH, init_body, 0)
    _zero_spmem_slices(zl_v, cnt_s, sid)
    plsc.subcore_barrier()

    def idx_start(c, b):
        i_v, isem = bufs[b]
        pltpu.async_copy(ei_h.at[0, pl.ds(base + c * CH, CH)], i_v, isem)

    def scatter(c, b):
        i_v, isem = bufs[b]
        pltpu.make_async_copy(ei_h.at[0, pl.ds(base + c * CH, CH)],
                              i_v, isem).wait()
        pltpu.sync_copy(ones_v, cnt_s.at[i_v], add=True)

    idx_start(0, 0)
    idx_start(1, 1)

    def iter_body(k, _):
        c = 2 * k
        scatter(c, 0)

        @pl.when(c + 2 < nch)
        def _():
            idx_start(c + 2, 0)

        scatter(c + 1, 1)

        @pl.when(c + 3 < nch)
        def _():
            idx_start(c + 3, 1)

        return 0

    lax.fori_loop(0, nch // 2, iter_body, 0)

    @pl.when(nch % 2 == 1)
    def _():
        scatter(nch - 1, 0)

    plsc.subcore_barrier()
    _write_spmem_slices(cnt_s, cnt_h, cid, sid)


@functools.lru_cache(maxsize=None)
def _sc_count_kernel():
    return pl.kernel(
        _sc_cnt_body,
        out_type=jax.ShapeDtypeStruct((NC, N, D), jnp.float32),
        mesh=_sc_mesh(),
        scratch_types=[
            pltpu.VMEM_SHARED((N, D), jnp.float32),
            pltpu.VMEM((CH,), jnp.int32),
            pltpu.VMEM((CH,), jnp.int32),
            pltpu.VMEM((CH, D), jnp.float32),
            pltpu.VMEM((CH, D), jnp.float32),
            pltpu.SemaphoreType.DMA,
            pltpu.SemaphoreType.DMA,
        ],
    )


def _tc_transform(x, w):
    bn = 1000

    def tbody(x_ref, w_ref, o_ref):
        o_ref[0] = jnp.dot(x_ref[...], w_ref[0],
                           preferred_element_type=jnp.float32)

    return pl.pallas_call(
        tbody,
        grid=(R, N // bn),
        in_specs=[pl.BlockSpec((bn, D), lambda r, n: (n, 0)),
                  pl.BlockSpec((1, D, D), lambda r, n: (r, 0, 0))],
        out_specs=pl.BlockSpec((1, bn, D), lambda r, n: (r, n, 0)),
        out_shape=jax.ShapeDtypeStruct((R, N, D), jnp.float32),
    )(x, w)


def _tc_norm_transform(p, cnt, w):
    bn = 1000

    def nbody(p_ref, c_ref, w_ref, o_ref):
        c = c_ref[0, :, 0:1] + c_ref[1, :, 0:1]
        h = jax.nn.sigmoid((p_ref[0] + p_ref[1]) / jnp.maximum(c, 1.0))
        for r in range(R):
            o_ref[r] = jnp.dot(h, w_ref[r], preferred_element_type=jnp.float32)

    return pl.pallas_call(
        nbody,
        grid=(N // bn,),
        in_specs=[pl.BlockSpec((2, bn, D), lambda n: (0, n, 0)),
                  pl.BlockSpec((2, bn, D), lambda n: (0, n, 0)),
                  pl.BlockSpec((R, D, D), lambda n: (0, 0, 0))],
        out_specs=pl.BlockSpec((R, bn, D), lambda n: (0, n, 0)),
        out_shape=jax.ShapeDtypeStruct((R, N, D), jnp.float32),
    )(p, cnt, w)


def _tc_head(p2, cnt, x, wl, bl, wf, bf):
    bn = 1000

    def hbody(p_ref, c_ref, x_ref, wl_ref, bl_ref, wf_ref, bf_ref,
              lp_ref, em_ref):
        c = c_ref[0, :, 0:1] + c_ref[1, :, 0:1]
        agg = (p_ref[0] + p_ref[1]) / jnp.maximum(c, 1.0)
        em = jnp.concatenate([x_ref[...], agg], axis=1)
        em_ref[...] = em
        h = lax.dot_general(em, wl_ref[...], (((1,), (1,)), ((), ())),
                            preferred_element_type=jnp.float32) + bl_ref[...]
        h = jnp.maximum(h, 0.0)
        logits = lax.dot_general(h, wf_ref[...], (((1,), (1,)), ((), ())),
                                 preferred_element_type=jnp.float32) + bf_ref[...]
        m = jnp.max(logits, axis=1, keepdims=True)
        z = logits - m
        lp_ref[...] = z - jnp.log(jnp.sum(jnp.exp(z), axis=1, keepdims=True))

    return pl.pallas_call(
        hbody,
        grid=(N // bn,),
        in_specs=[pl.BlockSpec((2, bn, D), lambda n: (0, n, 0)),
                  pl.BlockSpec((2, bn, D), lambda n: (0, n, 0)),
                  pl.BlockSpec((bn, D), lambda n: (n, 0)),
                  pl.BlockSpec((D, 2 * D), lambda n: (0, 0)),
                  pl.BlockSpec((1, D), lambda n: (0, 0)),
                  pl.BlockSpec((C, D), lambda n: (0, 0)),
                  pl.BlockSpec((1, C), lambda n: (0, 0))],
        out_specs=[pl.BlockSpec((bn, C), lambda n: (n, 0)),
                   pl.BlockSpec((bn, 2 * D), lambda n: (n, 0))],
        out_shape=[jax.ShapeDtypeStruct((N, C), jnp.float32),
                   jax.ShapeDtypeStruct((N, 2 * D), jnp.float32)],
    )(p2, cnt, x, wl, bl, wf, bf)


def kernel(speaker, x, edge_index, edge_norm, edge_type, seq_lengths, umask,
           w1, w2, W_lin, b_lin, W_fc, b_fc):
    ei = edge_index.astype(jnp.int32)
    et = edge_type.astype(jnp.int32)
    en = edge_norm.astype(jnp.float32)

    y1 = _tc_transform(x, w1).reshape(R * N, D)
    cnt = _sc_count_kernel()(ei)
    p1 = _sc_aggregate_kernel()(y1, ei, et, en)
    y2 = _tc_norm_transform(p1, cnt, w2).reshape(R * N, D)
    p2 = _sc_aggregate_kernel()(y2, ei, et, en)
    log_prob, emotions = _tc_head(p2, cnt, x, W_lin,
                                  b_lin.reshape(1, D), W_fc,
                                  b_fc.reshape(1, C))
    return (log_prob, x, emotions)


# speedup vs baseline: 15.3854x; 1.0803x over previous
import functools

import jax
import jax.numpy as jnp
from jax import lax
from jax.experimental import pallas as pl
from jax.experimental.pallas import tpu as pltpu
from jax.experimental.pallas import tpu_sc as plsc

N = 10000
E = 160000
D = 128
R = 8
C = 7

NC = 2
NS = 16
L = 16
NW = NC * NS
CH = 128
EPT = 5120
MAXCH = EPT // CH
ROWS_PT = 624


def _sc_mesh():
    return plsc.VectorSubcoreMesh(core_axis_name="c", subcore_axis_name="s")


def _zero_spmem_slices(src_v, dst_s, sid):
    r0 = sid * ROWS_PT
    ofs = 0
    while ofs < ROWS_PT:
        sz = min(CH, ROWS_PT - ofs)
        pltpu.sync_copy(src_v.at[pl.ds(0, sz)], dst_s.at[pl.ds(r0 + ofs, sz)])
        ofs += sz

    @pl.when(sid == NS - 1)
    def _():
        pltpu.sync_copy(src_v.at[pl.ds(0, N - NS * ROWS_PT)],
                        dst_s.at[pl.ds(NS * ROWS_PT, N - NS * ROWS_PT)])


def _write_spmem_slices(src_s, dst_h, cid, sid):
    r0 = sid * ROWS_PT
    pltpu.sync_copy(src_s.at[pl.ds(r0, ROWS_PT)],
                    dst_h.at[cid, pl.ds(r0, ROWS_PT)])

    @pl.when(sid == NS - 1)
    def _():
        pltpu.sync_copy(src_s.at[pl.ds(NS * ROWS_PT, N - NS * ROWS_PT)],
                        dst_h.at[cid, pl.ds(NS * ROWS_PT, N - NS * ROWS_PT)])


def _vcopy(src_v, dst_v):
    def body(k, _):
        s = pl.ds(k * L, L)
        dst_v[s] = src_v[s]
        return 0

    lax.fori_loop(0, CH // L, body, 0)


def _sc_agg_body(table_h, ei_h, et_h, en_h, out_h,
                 acc_s,
                 i0, j0, t0, g0, en0, r0, si0, sen0,
                 i1, j1, t1, g1, en1, r1, si1, sen1,
                 gs0, gs1, is0, is1):
    cid = lax.axis_index("c")
    sid = lax.axis_index("s")
    wid = sid * NC + cid
    base = wid * EPT
    nch = jnp.minimum((E - base) // CH, MAXCH)

    bufs = ((i0, j0, t0, g0, en0, r0, si0, sen0, gs0, is0),
            (i1, j1, t1, g1, en1, r1, si1, sen1, gs1, is1))

    def init_body(rr, _):
        z16 = jnp.zeros((L,), jnp.float32)
        for d8 in range(D // L):
            r0[rr, pl.ds(d8 * L, L)] = z16
        return 0

    lax.fori_loop(0, CH, init_body, 0)
    _zero_spmem_slices(r0, acc_s, sid)
    plsc.subcore_barrier()

    def idx_start(c, b):
        i_v, j_v, t_v, g_v, en_v, rows_v, si_v, sen_v, gsem, isem = bufs[b]
        off = base + c * CH
        pltpu.async_copy(ei_h.at[0, pl.ds(off, CH)], i_v, isem)
        pltpu.async_copy(ei_h.at[1, pl.ds(off, CH)], j_v, isem)
        pltpu.async_copy(et_h.at[pl.ds(off, CH)], t_v, isem)
        pltpu.async_copy(en_h.at[pl.ds(off, CH)], en_v, isem)

    def gather_start(c, b):
        i_v, j_v, t_v, g_v, en_v, rows_v, si_v, sen_v, gsem, isem = bufs[b]
        off = base + c * CH
        pltpu.make_async_copy(ei_h.at[0, pl.ds(off, CH)], i_v, isem).wait()
        pltpu.make_async_copy(ei_h.at[1, pl.ds(off, CH)], j_v, isem).wait()
        pltpu.make_async_copy(et_h.at[pl.ds(off, CH)], t_v, isem).wait()
        pltpu.make_async_copy(en_h.at[pl.ds(off, CH)], en_v, isem).wait()

        def gidx_body(k, _):
            s = pl.ds(k * L, L)
            g_v[s] = t_v[s] * N + j_v[s]
            return 0

        lax.fori_loop(0, CH // L, gidx_body, 0)
        pltpu.async_copy(table_h.at[g_v], rows_v, gsem)

    def save_idx(b):
        i_v, j_v, t_v, g_v, en_v, rows_v, si_v, sen_v, gsem, isem = bufs[b]
        _vcopy(i_v, si_v)
        _vcopy(en_v, sen_v)

    def scale_scatter(b):
        i_v, j_v, t_v, g_v, en_v, rows_v, si_v, sen_v, gsem, isem = bufs[b]
        pltpu.make_async_copy(table_h.at[g_v], rows_v, gsem).wait()

        def scale_body(k, _):
            env = sen_v[pl.ds(k * L, L)] * 2.0
            for lane in range(L):
                sv = jnp.full((L,), env[lane], jnp.float32)
                e = k * L + lane
                for d8 in range(D // L):
                    s = pl.ds(d8 * L, L)
                    rows_v[e, s] = rows_v[e, s] * sv
            return 0

        lax.fori_loop(0, CH // L, scale_body, 0)
        pltpu.sync_copy(rows_v, acc_s.at[si_v], add=True)

    idx_start(0, 0)
    idx_start(1, 1)
    gather_start(0, 0)

    def iter_body(k, _):
        c = 2 * k
        gather_start(c + 1, 1)
        save_idx(0)

        @pl.when(c + 2 < nch)
        def _():
            idx_start(c + 2, 0)

        scale_scatter(0)

        @pl.when(c + 2 < nch)
        def _():
            gather_start(c + 2, 0)

        save_idx(1)

        @pl.when(c + 3 < nch)
        def _():
            idx_start(c + 3, 1)

        scale_scatter(1)
        return 0

    lax.fori_loop(0, nch // 2, iter_body, 0)

    @pl.when(nch % 2 == 1)
    def _():
        save_idx(0)
        scale_scatter(0)

    plsc.subcore_barrier()
    _write_spmem_slices(acc_s, out_h, cid, sid)


@functools.lru_cache(maxsize=None)
def _sc_aggregate_kernel():
    scratch = [pltpu.VMEM_SHARED((N, D), jnp.float32)]
    for _ in range(2):
        scratch += [
            pltpu.VMEM((CH,), jnp.int32),
            pltpu.VMEM((CH,), jnp.int32),
            pltpu.VMEM((CH,), jnp.int32),
            pltpu.VMEM((CH,), jnp.int32),
            pltpu.VMEM((CH,), jnp.float32),
            pltpu.VMEM((CH, D), jnp.float32),
            pltpu.VMEM((CH,), jnp.int32),
            pltpu.VMEM((CH,), jnp.float32),
        ]
    scratch += [pltpu.SemaphoreType.DMA] * 4
    return pl.kernel(
        _sc_agg_body,
        out_type=jax.ShapeDtypeStruct((NC, N, D), jnp.float32),
        mesh=_sc_mesh(),
        scratch_types=scratch,
    )


def _sc_cnt_body(ei_h, cnt_h, cnt_s, i0, i1, ones_v, zl_v, is0, is1):
    cid = lax.axis_index("c")
    sid = lax.axis_index("s")
    wid = sid * NC + cid
    base = wid * EPT
    nch = jnp.minimum((E - base) // CH, MAXCH)

    bufs = ((i0, is0), (i1, is1))

    def init_body(rr, _):
        for d8 in range(D // L):
            s = pl.ds(d8 * L, L)
            ones_v[rr, s] = jnp.ones((L,), jnp.float32)
            zl_v[rr, s] = jnp.zeros((L,), jnp.float32)
        return 0

    lax.fori_loop(0, CH, init_body, 0)
    _zero_spmem_slices(zl_v, cnt_s, sid)
    plsc.subcore_barrier()

    def idx_start(c, b):
        i_v, isem = bufs[b]
        pltpu.async_copy(ei_h.at[0, pl.ds(base + c * CH, CH)], i_v, isem)

    def scatter(c, b):
        i_v, isem = bufs[b]
        pltpu.make_async_copy(ei_h.at[0, pl.ds(base + c * CH, CH)],
                              i_v, isem).wait()
        pltpu.sync_copy(ones_v, cnt_s.at[i_v], add=True)

    idx_start(0, 0)
    idx_start(1, 1)

    def iter_body(k, _):
        c = 2 * k
        scatter(c, 0)

        @pl.when(c + 2 < nch)
        def _():
            idx_start(c + 2, 0)

        scatter(c + 1, 1)

        @pl.when(c + 3 < nch)
        def _():
            idx_start(c + 3, 1)

        return 0

    lax.fori_loop(0, nch // 2, iter_body, 0)

    @pl.when(nch % 2 == 1)
    def _():
        scatter(nch - 1, 0)

    plsc.subcore_barrier()
    _write_spmem_slices(cnt_s, cnt_h, cid, sid)


@functools.lru_cache(maxsize=None)
def _sc_count_kernel():
    return pl.kernel(
        _sc_cnt_body,
        out_type=jax.ShapeDtypeStruct((NC, N, D), jnp.float32),
        mesh=_sc_mesh(),
        scratch_types=[
            pltpu.VMEM_SHARED((N, D), jnp.float32),
            pltpu.VMEM((CH,), jnp.int32),
            pltpu.VMEM((CH,), jnp.int32),
            pltpu.VMEM((CH, D), jnp.float32),
            pltpu.VMEM((CH, D), jnp.float32),
            pltpu.SemaphoreType.DMA,
            pltpu.SemaphoreType.DMA,
        ],
    )


def _tc_transform(x, w):
    bn = 1000

    def tbody(x_ref, w_ref, o_ref):
        for r in range(R):
            o_ref[r] = jnp.dot(x_ref[...], w_ref[r],
                               preferred_element_type=jnp.float32)

    return pl.pallas_call(
        tbody,
        grid=(N // bn,),
        in_specs=[pl.BlockSpec((bn, D), lambda n: (n, 0)),
                  pl.BlockSpec((R, D, D), lambda n: (0, 0, 0))],
        out_specs=pl.BlockSpec((R, bn, D), lambda n: (0, n, 0)),
        out_shape=jax.ShapeDtypeStruct((R, N, D), jnp.float32),
    )(x, w)


def _tc_norm_transform(p, cnt, w):
    bn = 1000

    def nbody(p_ref, c_ref, w_ref, o_ref):
        c = c_ref[0, :, 0:1] + c_ref[1, :, 0:1]
        h = jax.nn.sigmoid((p_ref[0] + p_ref[1]) / jnp.maximum(c, 1.0))
        for r in range(R):
            o_ref[r] = jnp.dot(h, w_ref[r], preferred_element_type=jnp.float32)

    return pl.pallas_call(
        nbody,
        grid=(N // bn,),
        in_specs=[pl.BlockSpec((2, bn, D), lambda n: (0, n, 0)),
                  pl.BlockSpec((2, bn, D), lambda n: (0, n, 0)),
                  pl.BlockSpec((R, D, D), lambda n: (0, 0, 0))],
        out_specs=pl.BlockSpec((R, bn, D), lambda n: (0, n, 0)),
        out_shape=jax.ShapeDtypeStruct((R, N, D), jnp.float32),
    )(p, cnt, w)


def _tc_head(p2, cnt, x, wl, bl, wf, bf):
    bn = 1000

    def hbody(p_ref, c_ref, x_ref, wl_ref, bl_ref, wf_ref, bf_ref,
              lp_ref, em_ref):
        c = c_ref[0, :, 0:1] + c_ref[1, :, 0:1]
        agg = (p_ref[0] + p_ref[1]) / jnp.maximum(c, 1.0)
        em = jnp.concatenate([x_ref[...], agg], axis=1)
        em_ref[...] = em
        h = lax.dot_general(em, wl_ref[...], (((1,), (1,)), ((), ())),
                            preferred_element_type=jnp.float32) + bl_ref[...]
        h = jnp.maximum(h, 0.0)
        logits = lax.dot_general(h, wf_ref[...], (((1,), (1,)), ((), ())),
                                 preferred_element_type=jnp.float32) + bf_ref[...]
        m = jnp.max(logits, axis=1, keepdims=True)
        z = logits - m
        lp_ref[...] = z - jnp.log(jnp.sum(jnp.exp(z), axis=1, keepdims=True))

    return pl.pallas_call(
        hbody,
        grid=(N // bn,),
        in_specs=[pl.BlockSpec((2, bn, D), lambda n: (0, n, 0)),
                  pl.BlockSpec((2, bn, D), lambda n: (0, n, 0)),
                  pl.BlockSpec((bn, D), lambda n: (n, 0)),
                  pl.BlockSpec((D, 2 * D), lambda n: (0, 0)),
                  pl.BlockSpec((1, D), lambda n: (0, 0)),
                  pl.BlockSpec((C, D), lambda n: (0, 0)),
                  pl.BlockSpec((1, C), lambda n: (0, 0))],
        out_specs=[pl.BlockSpec((bn, C), lambda n: (n, 0)),
                   pl.BlockSpec((bn, 2 * D), lambda n: (n, 0))],
        out_shape=[jax.ShapeDtypeStruct((N, C), jnp.float32),
                   jax.ShapeDtypeStruct((N, 2 * D), jnp.float32)],
    )(p2, cnt, x, wl, bl, wf, bf)


def kernel(speaker, x, edge_index, edge_norm, edge_type, seq_lengths, umask,
           w1, w2, W_lin, b_lin, W_fc, b_fc):
    ei = edge_index.astype(jnp.int32)
    et = edge_type.astype(jnp.int32)
    en = edge_norm.astype(jnp.float32)

    y1 = _tc_transform(x, w1).reshape(R * N, D)
    cnt = _sc_count_kernel()(ei)
    p1 = _sc_aggregate_kernel()(y1, ei, et, en)
    y2 = _tc_norm_transform(p1, cnt, w2).reshape(R * N, D)
    p2 = _sc_aggregate_kernel()(y2, ei, et, en)
    log_prob, emotions = _tc_head(p2, cnt, x, W_lin,
                                  b_lin.reshape(1, D), W_fc,
                                  b_fc.reshape(1, C))
    return (log_prob, x, emotions)
